# initial kernel scaffold (unmeasured)
import jax
import jax.numpy as jnp
from jax import lax
from jax.experimental import pallas as pl
from jax.experimental.pallas import tpu as pltpu


def kernel(
    x,
):
    def body(*refs):
        pass

    out_shape = jax.ShapeDtypeStruct(..., jnp.float32)
    return pl.pallas_call(body, out_shape=out_shape)(...)


# baseline (device time: 29972 ns/iter reference)
import jax
import jax.numpy as jnp
from jax import lax
from jax.experimental import pallas as pl
from jax.experimental.pallas import tpu as pltpu


def kernel(x):
    m, n = x.shape
    nh = n // 2

    def body(x_ref, out_ref, send_sem, recv_sem):
        my_x = lax.axis_index("x")
        my_y = lax.axis_index("y")
        my_z = lax.axis_index("z")
        other_y = 1 - my_y
        peer = (my_x, other_y, my_z)

        barrier = pltpu.get_barrier_semaphore()
        pl.semaphore_signal(
            barrier, inc=1, device_id=peer,
            device_id_type=pl.DeviceIdType.MESH,
        )
        pl.semaphore_wait(barrier, 1)

        rdma = pltpu.make_async_remote_copy(
            src_ref=x_ref.at[:, pl.ds(other_y * nh, nh)],
            dst_ref=out_ref.at[pl.ds(my_y * m, m), :],
            send_sem=send_sem,
            recv_sem=recv_sem,
            device_id=peer,
            device_id_type=pl.DeviceIdType.MESH,
        )
        rdma.start()

        out_ref[pl.ds(my_y * m, m), :] = x_ref[:, pl.ds(my_y * nh, nh)]

        rdma.wait()

    return pl.pallas_call(
        body,
        out_shape=jax.ShapeDtypeStruct((2 * m, nh), x.dtype),
        in_specs=[pl.BlockSpec(memory_space=pltpu.VMEM)],
        out_specs=pl.BlockSpec(memory_space=pltpu.VMEM),
        scratch_shapes=[
            pltpu.SemaphoreType.DMA,
            pltpu.SemaphoreType.DMA,
        ],
        compiler_params=pltpu.CompilerParams(collective_id=0),
    )(x)


# device time: 22958 ns/iter; 1.3055x vs baseline; 1.3055x over previous
import jax
import jax.numpy as jnp
from jax import lax
from jax.experimental import pallas as pl
from jax.experimental.pallas import tpu as pltpu

K = 8


def kernel(x):
    m, n = x.shape
    nh = n // 2
    half = m // 2
    blk = half // K

    def body(x_ref, out_ref, ysend_sems, yrecv_sems, xsend_sems, xrecv_sems):
        my_x = lax.axis_index("x")
        my_y = lax.axis_index("y")
        my_z = lax.axis_index("z")
        other_y = 1 - my_y
        other_x = 1 - my_x
        ypeer = (my_x, other_y, my_z)
        xpeer = (other_x, my_y, my_z)

        barrier = pltpu.get_barrier_semaphore()
        for p in (ypeer, xpeer):
            pl.semaphore_signal(
                barrier, inc=1, device_id=p,
                device_id_type=pl.DeviceIdType.MESH,
            )
        pl.semaphore_wait(barrier, 2)

        src_row0 = my_x * half
        dst_row0 = my_y * m + my_x * half
        fwd_row0 = other_y * m + my_x * half

        y_rdmas = []
        for k in range(K):
            r = pltpu.make_async_remote_copy(
                src_ref=x_ref.at[
                    pl.ds(src_row0 + k * blk, blk), pl.ds(other_y * nh, nh)
                ],
                dst_ref=out_ref.at[pl.ds(dst_row0 + k * blk, blk), :],
                send_sem=ysend_sems.at[k],
                recv_sem=yrecv_sems.at[k],
                device_id=ypeer,
                device_id_type=pl.DeviceIdType.MESH,
            )
            r.start()
            y_rdmas.append(r)

        out_ref[pl.ds(my_y * m, m), :] = x_ref[:, pl.ds(my_y * nh, nh)]

        x_rdmas = []
        for k in range(K):
            y_rdmas[k].wait_recv()
            r = pltpu.make_async_remote_copy(
                src_ref=out_ref.at[pl.ds(fwd_row0 + k * blk, blk), :],
                dst_ref=out_ref.at[pl.ds(fwd_row0 + k * blk, blk), :],
                send_sem=xsend_sems.at[k],
                recv_sem=xrecv_sems.at[k],
                device_id=xpeer,
                device_id_type=pl.DeviceIdType.MESH,
            )
            r.start()
            x_rdmas.append(r)

        for k in range(K):
            y_rdmas[k].wait_send()
            x_rdmas[k].wait_send()
            x_rdmas[k].wait_recv()

    return pl.pallas_call(
        body,
        out_shape=jax.ShapeDtypeStruct((2 * m, nh), x.dtype),
        in_specs=[pl.BlockSpec(memory_space=pltpu.VMEM)],
        out_specs=pl.BlockSpec(memory_space=pltpu.VMEM),
        scratch_shapes=[
            pltpu.SemaphoreType.DMA((K,)),
            pltpu.SemaphoreType.DMA((K,)),
            pltpu.SemaphoreType.DMA((K,)),
            pltpu.SemaphoreType.DMA((K,)),
        ],
        compiler_params=pltpu.CompilerParams(collective_id=0),
    )(x)
